# baseline (speedup 1.0000x reference)
import functools

import jax
import jax.numpy as jnp
from jax import lax
from jax.experimental import pallas as pl
from jax.experimental.pallas import tpu as pltpu
from jax.experimental.pallas import tpu_sc as plsc

DIM = 64
BATCH = 16384
_L = 16
_NC = 2
_NS = 16
_NW = _NC * _NS
_BPW = BATCH // _NW
_CH = 128
_NCHUNK = _BPW // _CH


def _proj_body(t_ref, w_ref, o_ref):
    o_ref[...] = lax.dot_general(
        t_ref[...], w_ref[...],
        dimension_numbers=(((1,), (1,)), ((), ())),
        preferred_element_type=jnp.float32,
        precision=lax.Precision.HIGHEST,
    )


def _time_proj(time_emb, W_time):
    return pl.pallas_call(
        _proj_body,
        out_shape=jax.ShapeDtypeStruct(time_emb.shape, jnp.float32),
    )(time_emb, W_time)


def _sc_body(subj_hbm, rel_idx_hbm, obj_hbm, time_idx_hbm,
             ent_hbm, rel_tab_hbm, proj_tab_hbm,
             es_out, er_out, eo_out, tp_out,
             idx_s, idx_r, idx_o, idx_t,
             rows_s, rows_o, rows_t, rows_r,
             sem_s, sem_o, sem_t, sem_r):
    wid = lax.axis_index("s") * _NC + lax.axis_index("c")
    base = wid * _BPW
    pltpu.sync_copy(subj_hbm.at[pl.ds(base, _BPW)], idx_s)
    pltpu.sync_copy(rel_idx_hbm.at[pl.ds(base, _BPW)], idx_r)
    pltpu.sync_copy(obj_hbm.at[pl.ds(base, _BPW)], idx_o)
    pltpu.sync_copy(time_idx_hbm.at[pl.ds(base, _BPW)], idx_t)
    for c in range(_NCHUNK):
        isl = pl.ds(c * _CH, _CH)
        osl = pl.ds(base + c * _CH, _CH)
        cp_t = pltpu.async_copy(proj_tab_hbm.at[idx_t.at[isl]], rows_t, sem_t)
        cp_s = pltpu.async_copy(ent_hbm.at[idx_s.at[isl]], rows_s, sem_s)
        cp_o = pltpu.async_copy(ent_hbm.at[idx_o.at[isl]], rows_o, sem_o)
        cp_r = pltpu.async_copy(rel_tab_hbm.at[idx_r.at[isl]], rows_r, sem_r)
        cp_r.wait()
        pltpu.sync_copy(rows_r, er_out.at[osl])
        cp_t.wait()
        pltpu.sync_copy(rows_t, tp_out.at[osl])
        cp_s.wait()
        cp_o.wait()

        def add_row(r, _):
            for j in range(DIM // _L):
                sl = pl.ds(j * _L, _L)
                tv = rows_t[r, sl]
                rows_s[r, sl] = rows_s[r, sl] + tv
                rows_o[r, sl] = rows_o[r, sl] + tv
            return 0

        lax.fori_loop(0, _CH, add_row, 0)
        pltpu.sync_copy(rows_s, es_out.at[osl])
        pltpu.sync_copy(rows_o, eo_out.at[osl])


@jax.jit
def kernel(subjects, relations, objects, time_ids, ent_emb, rel_emb, time_emb,
           W_time):
    proj_tab = _time_proj(time_emb, W_time)
    out_t = jax.ShapeDtypeStruct((BATCH, DIM), jnp.float32)
    mesh = plsc.VectorSubcoreMesh(core_axis_name="c", subcore_axis_name="s",
                                  num_cores=_NC, num_subcores=_NS)
    f = pl.kernel(
        _sc_body,
        out_type=[out_t, out_t, out_t, out_t],
        mesh=mesh,
        compiler_params=pltpu.CompilerParams(use_tc_tiling_on_sc=False),
        scratch_types=[
            pltpu.VMEM((_BPW,), jnp.int32),
            pltpu.VMEM((_BPW,), jnp.int32),
            pltpu.VMEM((_BPW,), jnp.int32),
            pltpu.VMEM((_BPW,), jnp.int32),
            pltpu.VMEM((_CH, DIM), jnp.float32),
            pltpu.VMEM((_CH, DIM), jnp.float32),
            pltpu.VMEM((_CH, DIM), jnp.float32),
            pltpu.VMEM((_CH, DIM), jnp.float32),
            pltpu.SemaphoreType.DMA,
            pltpu.SemaphoreType.DMA,
            pltpu.SemaphoreType.DMA,
            pltpu.SemaphoreType.DMA,
        ],
    )
    e_s, e_r, e_o, t_proj = f(subjects, relations, objects, time_ids,
                              ent_emb, rel_emb, proj_tab)
    return (e_s, e_r, e_o, t_proj)


# speedup vs baseline: 1.8891x; 1.8891x over previous
import functools

import jax
import jax.numpy as jnp
from jax import lax
from jax.experimental import pallas as pl
from jax.experimental.pallas import tpu as pltpu
from jax.experimental.pallas import tpu_sc as plsc

DIM = 64
BATCH = 16384
NENT = 1000000
NTAB = 1000
_L = 16
_NC = 2
_NS = 16
_NW = _NC * _NS
_BPW = BATCH // _NW
_CH = 32
_NCHUNK = _BPW // _CH


def _proj_body(t_ref, w_ref, o_ref):
    res = lax.dot_general(
        t_ref[...], w_ref[...],
        dimension_numbers=(((1,), (1,)), ((), ())),
        preferred_element_type=jnp.float32,
        precision=lax.Precision.HIGHEST,
    )
    o_ref[...] = jnp.concatenate([res, jnp.zeros_like(res)], axis=1)


def _time_proj(time_emb, W_time):
    return pl.pallas_call(
        _proj_body,
        out_shape=jax.ShapeDtypeStruct((NTAB, 2 * DIM), jnp.float32),
    )(time_emb, W_time)


def _sc_body(subj_hbm, rel_idx_hbm, obj_hbm, time_idx_hbm,
             ent3_hbm, rel128_hbm, proj128_hbm,
             es_out, er_out, eo_out, tp_out,
             idx_s, idx_o, idx_r, idx_t,
             ring_s, ring_o, rows_t, rows_r, stage,
             ks_smem, ko_smem,
             sem_s, sem_o, sem_t, sem_r):
    wid = lax.axis_index("s") * _NC + lax.axis_index("c")
    base = wid * _BPW
    bsl = pl.ds(base, _BPW)
    pltpu.sync_copy(subj_hbm.at[bsl], idx_s)
    pltpu.sync_copy(obj_hbm.at[bsl], idx_o)
    pltpu.sync_copy(rel_idx_hbm.at[bsl], idx_r)
    pltpu.sync_copy(time_idx_hbm.at[bsl], idx_t)

    lanes = lax.iota(jnp.int32, _L)

    def idx_to_smem(i, _):
        vs = idx_s[pl.ds(i * _L, _L)]
        vo = idx_o[pl.ds(i * _L, _L)]
        for lane in range(_L):
            onehot = lanes == lane
            ks_smem[i * _L + lane] = jnp.sum(jnp.where(onehot, vs, 0))
            ko_smem[i * _L + lane] = jnp.sum(jnp.where(onehot, vo, 0))
        return 0

    lax.fori_loop(0, _BPW // _L, idx_to_smem, 0)

    for c in range(_NCHUNK):
        isl = pl.ds(c * _CH, _CH)
        osl = pl.ds(base + c * _CH, _CH)
        cp_t = pltpu.async_copy(proj128_hbm.at[idx_t.at[isl]], rows_t, sem_t)
        cp_r = pltpu.async_copy(rel128_hbm.at[idx_r.at[isl]], rows_r, sem_r)

        def issue(r, _):
            gs = lax.shift_right_logical(ks_smem[c * _CH + r], 3)
            go = lax.shift_right_logical(ko_smem[c * _CH + r], 3)
            pltpu.async_copy(ent3_hbm.at[gs], ring_s.at[r], sem_s)
            pltpu.async_copy(ent3_hbm.at[go], ring_o.at[r], sem_o)
            return 0

        lax.fori_loop(0, _CH, issue, 0)

        g0 = (base + c * _CH) // 8
        gsl = pl.ds(g0, _CH // 8)

        cp_r.wait()

        def pk_r(r, _):
            for j in range(DIM // _L):
                sl16 = pl.ds(j * _L, _L)
                stage[r >> 3, r & 7, sl16] = rows_r[r, sl16]
            return 0

        lax.fori_loop(0, _CH, pk_r, 0)
        pltpu.sync_copy(stage, er_out.at[gsl])

        cp_t.wait()

        def pk_t(r, _):
            for j in range(DIM // _L):
                sl16 = pl.ds(j * _L, _L)
                stage[r >> 3, r & 7, sl16] = rows_t[r, sl16]
            return 0

        lax.fori_loop(0, _CH, pk_t, 0)
        pltpu.sync_copy(stage, tp_out.at[gsl])

        pltpu.make_async_copy(ent3_hbm.at[pl.ds(0, _CH)], ring_s, sem_s).wait()

        def ex_s(r, _):
            k = ks_smem[c * _CH + r] & 7
            for j in range(DIM // _L):
                sl16 = pl.ds(j * _L, _L)
                stage[r >> 3, r & 7, sl16] = ring_s[r, k, sl16] + rows_t[r, sl16]
            return 0

        lax.fori_loop(0, _CH, ex_s, 0)
        pltpu.sync_copy(stage, es_out.at[gsl])

        pltpu.make_async_copy(ent3_hbm.at[pl.ds(0, _CH)], ring_o, sem_o).wait()

        def ex_o(r, _):
            k = ko_smem[c * _CH + r] & 7
            for j in range(DIM // _L):
                sl16 = pl.ds(j * _L, _L)
                stage[r >> 3, r & 7, sl16] = ring_o[r, k, sl16] + rows_t[r, sl16]
            return 0

        lax.fori_loop(0, _CH, ex_o, 0)
        pltpu.sync_copy(stage, eo_out.at[gsl])


@jax.jit
def kernel(subjects, relations, objects, time_ids, ent_emb, rel_emb, time_emb,
           W_time):
    proj128 = _time_proj(time_emb, W_time)
    rel128 = jnp.pad(rel_emb, ((0, 0), (0, DIM)))
    ent3 = ent_emb.reshape(NENT // 8, 8, DIM)
    out_t = jax.ShapeDtypeStruct((BATCH // 8, 8, DIM), jnp.float32)
    mesh = plsc.VectorSubcoreMesh(core_axis_name="c", subcore_axis_name="s",
                                  num_cores=_NC, num_subcores=_NS)
    f = pl.kernel(
        _sc_body,
        out_type=[out_t, out_t, out_t, out_t],
        mesh=mesh,
        compiler_params=pltpu.CompilerParams(needs_layout_passes=False),
        scratch_types=[
            pltpu.VMEM((_BPW,), jnp.int32),
            pltpu.VMEM((_BPW,), jnp.int32),
            pltpu.VMEM((_BPW,), jnp.int32),
            pltpu.VMEM((_BPW,), jnp.int32),
            pltpu.VMEM((_CH, 8, DIM), jnp.float32),
            pltpu.VMEM((_CH, 8, DIM), jnp.float32),
            pltpu.VMEM((_CH, 2 * DIM), jnp.float32),
            pltpu.VMEM((_CH, 2 * DIM), jnp.float32),
            pltpu.VMEM((_CH // 8, 8, DIM), jnp.float32),
            pltpu.SMEM((_BPW,), jnp.int32),
            pltpu.SMEM((_BPW,), jnp.int32),
            pltpu.SemaphoreType.DMA,
            pltpu.SemaphoreType.DMA,
            pltpu.SemaphoreType.DMA,
            pltpu.SemaphoreType.DMA,
        ],
    )
    e_s, e_r, e_o, t_proj = f(subjects, relations, objects, time_ids,
                              ent3, rel128, proj128)
    shp = (BATCH, DIM)
    return (e_s.reshape(shp), e_r.reshape(shp), e_o.reshape(shp),
            t_proj.reshape(shp))
